# baseline (speedup 1.0000x reference)
import functools

import jax
import jax.numpy as jnp
from jax import lax
from jax.experimental import pallas as pl
from jax.experimental.pallas import tpu as pltpu
from jax.experimental.pallas import tpu_sc as plsc

_CFGS = [
    (1024, 32, 64, 2), (1024, 64, 64, 4), (1024, 64, 128, 2),
    (1024, 128, 128, 4), (256, 128, 256, 2), (256, 256, 256, 4),
    (64, 256, 512, 2), (64, 512, 512, 4),
]
_K = 20
_NC, _NS = 2, 16
_NW = _NC * _NS


def _knn_body(pts_ref, idx_ref, *, n, k):
    x = pts_ref[0]
    sq = jnp.sum(x * x, axis=1)
    inner = jax.lax.dot_general(
        x, x, (((1,), (1,)), ((), ())),
        preferred_element_type=jnp.float32)
    dist = (sq[:, None] - 2.0 * inner) + sq[None, :]
    iota = jax.lax.broadcasted_iota(jnp.int32, (n, n), 1)
    for kk in range(k):
        m = jnp.min(dist, axis=1, keepdims=True)
        eq = dist == m
        cand = jnp.where(eq, iota, n)
        sel = jnp.min(cand, axis=1)
        idx_ref[0, kk, :] = sel
        dist = jnp.where(iota == sel[:, None], 1e30, dist)


def _knn(pts, k):
    b, n, _ = pts.shape
    return pl.pallas_call(
        functools.partial(_knn_body, n=n, k=k),
        grid=(b,),
        in_specs=[pl.BlockSpec((1, n, 3), lambda i: (i, 0, 0))],
        out_specs=pl.BlockSpec((1, k, n), lambda i: (i, 0, 0)),
        out_shape=jax.ShapeDtypeStruct((b, k, n), jnp.int32),
    )(pts)


def _lpfa_body(pts_ref, idx_ref, wsum_ref, wdiff_ref, out_ref, *, n, k):
    x = pts_ref[0]
    p = x @ wsum_ref[...]
    v = x @ wdiff_ref[...]
    iota = jax.lax.broadcasted_iota(jnp.int32, (n, n), 1)
    m = None
    for kk in range(k):
        sel = idx_ref[0, kk, :]
        oh = (sel[:, None] == iota).astype(jnp.float32)
        g = jax.lax.dot_general(
            oh, p, (((1,), (0,)), ((), ())),
            preferred_element_type=jnp.float32)
        m = g if m is None else jnp.maximum(m, g)
    out_ref[0] = jax.nn.relu(m + v)


def _lpfa(pts, idx, wsum, wdiff):
    b, n, _ = pts.shape
    c = wsum.shape[1]
    return pl.pallas_call(
        functools.partial(_lpfa_body, n=n, k=_K),
        grid=(b,),
        in_specs=[
            pl.BlockSpec((1, n, 3), lambda i: (i, 0, 0)),
            pl.BlockSpec((1, _K, n), lambda i: (i, 0, 0)),
            pl.BlockSpec(wsum.shape, lambda i: (0, 0)),
            pl.BlockSpec(wdiff.shape, lambda i: (0, 0)),
        ],
        out_specs=pl.BlockSpec((1, n, c), lambda i: (i, 0, 0)),
        out_shape=jax.ShapeDtypeStruct((b, n, c), jnp.float32),
    )(pts, idx, wsum, wdiff)


def _gmax_body(u_hbm, idx_hbm, out_hbm, idx_v, *bufs,
               n_chunks, c_out, np_, pk, nbuf):
    rb = bufs[:nbuf]
    ob = bufs[nbuf:2 * nbuf]
    sr = bufs[2 * nbuf:3 * nbuf]
    so = bufs[3 * nbuf:4 * nbuf]
    wid = lax.axis_index("s") * _NC + lax.axis_index("c")
    chunk0 = wid * n_chunks
    row0 = chunk0 * np_
    pltpu.sync_copy(idx_hbm.at[pl.ds(chunk0, n_chunks)], idx_v)
    for j in range(nbuf - 1):
        pltpu.async_copy(u_hbm.at[idx_v.at[j]], rb[j], sr[j])

    def step(i, _):
        for par in range(nbuf):
            g = i * nbuf + par
            nb = (par + nbuf - 1) % nbuf
            pltpu.make_async_copy(
                u_hbm.at[pl.ds(0, pk)], rb[par], sr[par]).wait()

            @pl.when(g + nbuf - 1 < n_chunks)
            def _():
                pltpu.async_copy(
                    u_hbm.at[idx_v.at[g + nbuf - 1]], rb[nb], sr[nb])

            @pl.when(g >= nbuf)
            def _():
                pltpu.make_async_copy(
                    ob[par], out_hbm.at[pl.ds(0, np_)], so[par]).wait()

            for p in range(np_):
                for cc in range(c_out // 16):
                    sl = pl.ds(cc * 16, 16)
                    acc = rb[par][p * _K, sl]
                    for kk in range(1, _K):
                        acc = jnp.maximum(acc, rb[par][p * _K + kk, sl])
                    ob[par][p, sl] = acc
            pltpu.async_copy(
                ob[par], out_hbm.at[pl.ds(row0 + g * np_, np_)], so[par])
        return 0

    lax.fori_loop(0, n_chunks // nbuf, step, 0)
    for j in range(nbuf):
        pltpu.make_async_copy(ob[j], out_hbm.at[pl.ds(0, np_)], so[j]).wait()


def _gmax(table, idx_chunks, c_out):
    nt, ct = table.shape
    nc_total, pk = idx_chunks.shape
    np_ = pk // _K
    n_chunks = nc_total // _NW
    mesh = plsc.VectorSubcoreMesh(core_axis_name="c", subcore_axis_name="s",
                                  num_cores=_NC, num_subcores=_NS)
    nbuf = 2 if ct >= 512 else 4
    f = pl.kernel(
        functools.partial(_gmax_body, n_chunks=n_chunks, c_out=c_out,
                          np_=np_, pk=pk, nbuf=nbuf),
        out_type=jax.ShapeDtypeStruct((nc_total * np_, c_out), jnp.float32),
        mesh=mesh,
        scratch_types=(
            [pltpu.VMEM((n_chunks, pk), jnp.int32)]
            + [pltpu.VMEM((pk, ct), jnp.float32)] * nbuf
            + [pltpu.VMEM((np_, c_out), jnp.float32)] * nbuf
            + [pltpu.SemaphoreType.DMA] * (2 * nbuf)
        ),
    )
    return f(table, idx_chunks)


def _tables_body(m_ref, v_ref, s_ref, w1_ref, w2a_ref, w2d_ref, wsc_ref,
                 u_ref, vo_ref, so_ref, *, mode, upad):
    if mode == 'feat':
        f = m_ref[...]
    else:
        f = jax.nn.relu(jax.nn.relu(m_ref[...] + v_ref[...]) + s_ref[...])
    h = jax.nn.relu(f @ w1_ref[...])
    u = h @ w2a_ref[...]
    if upad:
        u = jnp.concatenate(
            [u, jnp.zeros((u.shape[0], upad), u.dtype)], axis=1)
    u_ref[...] = u
    vo_ref[...] = h @ w2d_ref[...]
    so_ref[...] = f @ wsc_ref[...]


def _tables(m, v, s, w1, w2a, w2d, wsc, r=2048):
    rows, cin = m.shape[0], w1.shape[0]
    cout = wsc.shape[1]
    ct = max(cout, 128)
    upad = ct - cout
    mode = 'feat' if v is None else 'mid'
    ins = (m,) if v is None else (m, v, s)
    r = min(r, rows)
    specs = [pl.BlockSpec((r, cin), lambda i: (i, 0))] * len(ins)
    specs += [pl.BlockSpec(w.shape, lambda i: (0, 0))
              for w in (w1, w2a, w2d, wsc)]
    if mode == 'feat':
        body = lambda m_, *a: _tables_body(m_, None, None, *a,
                                           mode='feat', upad=upad)
    else:
        body = functools.partial(_tables_body, mode='mid', upad=upad)
    return pl.pallas_call(
        body,
        grid=(rows // r,),
        in_specs=specs,
        out_specs=[
            pl.BlockSpec((r, ct), lambda i: (i, 0)),
            pl.BlockSpec((r, cout), lambda i: (i, 0)),
            pl.BlockSpec((r, cout), lambda i: (i, 0)),
        ],
        out_shape=[
            jax.ShapeDtypeStruct((rows, ct), jnp.float32),
            jax.ShapeDtypeStruct((rows, cout), jnp.float32),
            jax.ShapeDtypeStruct((rows, cout), jnp.float32),
        ],
    )(*ins, w1, w2a, w2d, wsc)


def _head_body(m_ref, v_ref, s_ref, c0_ref, c1_ref, c2_ref, b2_ref,
               logits_ref, latent_ref):
    f = jax.nn.relu(jax.nn.relu(m_ref[...] + v_ref[...]) + s_ref[...])
    b, n, c = f.shape
    h = jax.nn.relu(jnp.reshape(f, (b * n, c)) @ c0_ref[...])
    h = jnp.reshape(h, (b, n, h.shape[1]))
    mx = jnp.max(h, axis=1)
    av = jnp.sum(h, axis=1) * (1.0 / n)
    latent = jnp.concatenate([mx, av], axis=1)
    x1 = jax.nn.relu(latent @ c1_ref[...])
    logits_ref[...] = x1 @ c2_ref[...] + b2_ref[...]
    latent_ref[...] = latent


def _head(m, v, s, c0, c1, c2, b2):
    b = m.shape[0]
    return pl.pallas_call(
        _head_body,
        out_shape=(
            jax.ShapeDtypeStruct((b, c2.shape[1]), jnp.float32),
            jax.ShapeDtypeStruct((b, 2 * c0.shape[1]), jnp.float32),
        ),
    )(m, v, s, c0, c1, c2, b2.reshape(1, -1))


def _global_idx(idx_t, n, b):
    g = jnp.swapaxes(idx_t, 1, 2)
    return g + (jnp.arange(b, dtype=jnp.int32) * n)[:, None, None]


def _chunked(idx_g, np_):
    return idx_g.reshape(-1, np_ * _K)


def _sub4(x, b, n):
    return x.reshape(b, n, -1)[:, ::4].reshape(b * n // 4, -1)


def kernel(xyz, params):
    b = xyz.shape[0]
    pts_full = jnp.swapaxes(xyz, 1, 2)
    h = b // 2
    halves = [pts_full[:h], pts_full[h:]]

    lw = params['lpfa_W']
    wsum = lw[0:3] + lw[3:6]
    wdiff = lw[6:9] - lw[0:3]

    def _w(p, cout, ratio):
        mid = cout // ratio
        return p['W1'], p['W2'][:mid], p['W2'][mid:] - p['W2'][:mid], p['Wsc']

    st = []
    for pts in halves:
        idxt1024 = _knn(pts, _K)
        idx1024 = _global_idx(idxt1024, 1024, h)
        idx256 = _global_idx(_knn(pts[:, ::4], _K), 256, h)
        idx64 = _global_idx(_knn(pts[:, ::16], _K), 64, h)
        chunks_for = {1024: _chunked(idx1024, 4),
                      256: _chunked(idx256, 2),
                      64: _chunked(idx64, 2)}
        sub_for = {1024: _chunked(idx1024[:, ::4], 4),
                   256: _chunked(idx256[:, ::4], 2)}
        feat = _lpfa(pts, idxt1024, wsum, wdiff)
        st.append(dict(chunks=chunks_for, subs=sub_for, cur_n=1024,
                       m=feat.reshape(h * 1024, 32), v=None, s=None))

    for bi in range(len(_CFGS)):
        (npoint, cin, cout, ratio), p = _CFGS[bi], params['cic'][bi]
        w1, w2a, w2d, wsc = _w(p, cout, ratio)
        nxt = _CFGS[bi + 1][0] if bi + 1 < len(_CFGS) else None
        for t in st:
            t['uvs'] = _tables(t['m'], t['v'], t['s'], w1, w2a, w2d, wsc)
        for t in st:
            u, vn, sn = t['uvs']
            if nxt is not None and nxt < t['cur_n']:
                t['m'] = _gmax(u, t['subs'][t['cur_n']], cout)
                t['v'] = _sub4(vn, h, t['cur_n'])
                t['s'] = _sub4(sn, h, t['cur_n'])
                t['cur_n'] = nxt
            else:
                t['m'] = _gmax(u, t['chunks'][t['cur_n']], cout)
                t['v'], t['s'] = vn, sn

    c0 = params['conv0_W']
    outs = [_head(t['m'].reshape(h, 64, 512), t['v'].reshape(h, 64, 512),
                  t['s'].reshape(h, 64, 512), c0, params['conv1_W'],
                  params['conv2_W'], params['conv2_b']) for t in st]
    return (jnp.concatenate([outs[0][0], outs[1][0]], axis=0),
            jnp.concatenate([outs[0][1], outs[1][1]], axis=0))


# speedup vs baseline: 1.0123x; 1.0123x over previous
import functools

import jax
import jax.numpy as jnp
from jax import lax
from jax.experimental import pallas as pl
from jax.experimental.pallas import tpu as pltpu
from jax.experimental.pallas import tpu_sc as plsc

_CFGS = [
    (1024, 32, 64, 2), (1024, 64, 64, 4), (1024, 64, 128, 2),
    (1024, 128, 128, 4), (256, 128, 256, 2), (256, 256, 256, 4),
    (64, 256, 512, 2), (64, 512, 512, 4),
]
_K = 20
_NC, _NS = 2, 16
_NW = _NC * _NS


def _knn_body(dist_ref, idx_ref, *, n, k):
    dist = dist_ref[0]
    iota = jax.lax.broadcasted_iota(jnp.int32, (n, n), 1)
    for kk in range(k):
        m = jnp.min(dist, axis=1, keepdims=True)
        eq = dist == m
        cand = jnp.where(eq, iota, n)
        sel = jnp.min(cand, axis=1)
        idx_ref[0, kk, :] = sel
        dist = jnp.where(iota == sel[:, None], 1e30, dist)


def _knn(pts, k):
    b, n, _ = pts.shape
    sq = jnp.sum(pts * pts, axis=-1)
    inner = jnp.einsum('bnc,bmc->bnm', pts, pts)
    dist = sq[:, :, None] - 2.0 * inner + sq[:, None, :]
    return pl.pallas_call(
        functools.partial(_knn_body, n=n, k=k),
        grid=(b,),
        in_specs=[pl.BlockSpec((1, n, n), lambda i: (i, 0, 0))],
        out_specs=pl.BlockSpec((1, k, n), lambda i: (i, 0, 0)),
        out_shape=jax.ShapeDtypeStruct((b, k, n), jnp.int32),
    )(dist)


def _lpfa_body(pts_ref, idx_ref, wsum_ref, wdiff_ref, out_ref, *, n, k):
    x = pts_ref[0]
    p = x @ wsum_ref[...]
    v = x @ wdiff_ref[...]
    iota = jax.lax.broadcasted_iota(jnp.int32, (n, n), 1)
    m = None
    for kk in range(k):
        sel = idx_ref[0, kk, :]
        oh = (sel[:, None] == iota).astype(jnp.float32)
        g = jax.lax.dot_general(
            oh, p, (((1,), (0,)), ((), ())),
            preferred_element_type=jnp.float32)
        m = g if m is None else jnp.maximum(m, g)
    out_ref[0] = jax.nn.relu(m + v)


def _lpfa(pts, idx, wsum, wdiff):
    b, n, _ = pts.shape
    c = wsum.shape[1]
    return pl.pallas_call(
        functools.partial(_lpfa_body, n=n, k=_K),
        grid=(b,),
        in_specs=[
            pl.BlockSpec((1, n, 3), lambda i: (i, 0, 0)),
            pl.BlockSpec((1, _K, n), lambda i: (i, 0, 0)),
            pl.BlockSpec(wsum.shape, lambda i: (0, 0)),
            pl.BlockSpec(wdiff.shape, lambda i: (0, 0)),
        ],
        out_specs=pl.BlockSpec((1, n, c), lambda i: (i, 0, 0)),
        out_shape=jax.ShapeDtypeStruct((b, n, c), jnp.float32),
    )(pts, idx, wsum, wdiff)


def _gmax_body(u_hbm, idx_hbm, out_hbm, idx_v, *bufs,
               n_chunks, c_out, np_, pk, nbuf):
    rb = bufs[:nbuf]
    ob = bufs[nbuf:2 * nbuf]
    sr = bufs[2 * nbuf:3 * nbuf]
    so = bufs[3 * nbuf:4 * nbuf]
    wid = lax.axis_index("s") * _NC + lax.axis_index("c")
    chunk0 = wid * n_chunks
    row0 = chunk0 * np_
    pltpu.sync_copy(idx_hbm.at[pl.ds(chunk0, n_chunks)], idx_v)
    for j in range(nbuf - 1):
        pltpu.async_copy(u_hbm.at[idx_v.at[j]], rb[j], sr[j])

    def step(i, _):
        for par in range(nbuf):
            g = i * nbuf + par
            nb = (par + nbuf - 1) % nbuf
            pltpu.make_async_copy(
                u_hbm.at[pl.ds(0, pk)], rb[par], sr[par]).wait()

            @pl.when(g + nbuf - 1 < n_chunks)
            def _():
                pltpu.async_copy(
                    u_hbm.at[idx_v.at[g + nbuf - 1]], rb[nb], sr[nb])

            @pl.when(g >= nbuf)
            def _():
                pltpu.make_async_copy(
                    ob[par], out_hbm.at[pl.ds(0, np_)], so[par]).wait()

            for p in range(np_):
                for cc in range(c_out // 16):
                    sl = pl.ds(cc * 16, 16)
                    acc = rb[par][p * _K, sl]
                    for kk in range(1, _K):
                        acc = jnp.maximum(acc, rb[par][p * _K + kk, sl])
                    ob[par][p, sl] = acc
            pltpu.async_copy(
                ob[par], out_hbm.at[pl.ds(row0 + g * np_, np_)], so[par])
        return 0

    lax.fori_loop(0, n_chunks // nbuf, step, 0)
    for j in range(nbuf):
        pltpu.make_async_copy(ob[j], out_hbm.at[pl.ds(0, np_)], so[j]).wait()


def _gmax(table, idx_chunks, c_out):
    nt, ct = table.shape
    nc_total, pk = idx_chunks.shape
    np_ = pk // _K
    n_chunks = nc_total // _NW
    mesh = plsc.VectorSubcoreMesh(core_axis_name="c", subcore_axis_name="s",
                                  num_cores=_NC, num_subcores=_NS)
    nbuf = 2 if ct >= 512 else 4
    f = pl.kernel(
        functools.partial(_gmax_body, n_chunks=n_chunks, c_out=c_out,
                          np_=np_, pk=pk, nbuf=nbuf),
        out_type=jax.ShapeDtypeStruct((nc_total * np_, c_out), jnp.float32),
        mesh=mesh,
        scratch_types=(
            [pltpu.VMEM((n_chunks, pk), jnp.int32)]
            + [pltpu.VMEM((pk, ct), jnp.float32)] * nbuf
            + [pltpu.VMEM((np_, c_out), jnp.float32)] * nbuf
            + [pltpu.SemaphoreType.DMA] * (2 * nbuf)
        ),
    )
    return f(table, idx_chunks)


def _tables_body(m_ref, v_ref, s_ref, w1_ref, w2a_ref, w2d_ref, wsc_ref,
                 u_ref, vo_ref, so_ref, *, mode, upad):
    if mode == 'feat':
        f = m_ref[...]
    else:
        f = jax.nn.relu(jax.nn.relu(m_ref[...] + v_ref[...]) + s_ref[...])
    h = jax.nn.relu(f @ w1_ref[...])
    u = h @ w2a_ref[...]
    if upad:
        u = jnp.concatenate(
            [u, jnp.zeros((u.shape[0], upad), u.dtype)], axis=1)
    u_ref[...] = u
    vo_ref[...] = h @ w2d_ref[...]
    so_ref[...] = f @ wsc_ref[...]


def _tables(m, v, s, w1, w2a, w2d, wsc, r=2048):
    rows, cin = m.shape[0], w1.shape[0]
    cout = wsc.shape[1]
    ct = max(cout, 128)
    upad = ct - cout
    mode = 'feat' if v is None else 'mid'
    ins = (m,) if v is None else (m, v, s)
    r = min(r, rows)
    specs = [pl.BlockSpec((r, cin), lambda i: (i, 0))] * len(ins)
    specs += [pl.BlockSpec(w.shape, lambda i: (0, 0))
              for w in (w1, w2a, w2d, wsc)]
    if mode == 'feat':
        body = lambda m_, *a: _tables_body(m_, None, None, *a,
                                           mode='feat', upad=upad)
    else:
        body = functools.partial(_tables_body, mode='mid', upad=upad)
    return pl.pallas_call(
        body,
        grid=(rows // r,),
        in_specs=specs,
        out_specs=[
            pl.BlockSpec((r, ct), lambda i: (i, 0)),
            pl.BlockSpec((r, cout), lambda i: (i, 0)),
            pl.BlockSpec((r, cout), lambda i: (i, 0)),
        ],
        out_shape=[
            jax.ShapeDtypeStruct((rows, ct), jnp.float32),
            jax.ShapeDtypeStruct((rows, cout), jnp.float32),
            jax.ShapeDtypeStruct((rows, cout), jnp.float32),
        ],
    )(*ins, w1, w2a, w2d, wsc)


def _head_body(m_ref, v_ref, s_ref, c0_ref, c1_ref, c2_ref, b2_ref,
               logits_ref, latent_ref):
    f = jax.nn.relu(jax.nn.relu(m_ref[...] + v_ref[...]) + s_ref[...])
    b, n, c = f.shape
    h = jax.nn.relu(jnp.reshape(f, (b * n, c)) @ c0_ref[...])
    h = jnp.reshape(h, (b, n, h.shape[1]))
    mx = jnp.max(h, axis=1)
    av = jnp.sum(h, axis=1) * (1.0 / n)
    latent = jnp.concatenate([mx, av], axis=1)
    x1 = jax.nn.relu(latent @ c1_ref[...])
    logits_ref[...] = x1 @ c2_ref[...] + b2_ref[...]
    latent_ref[...] = latent


def _head(m, v, s, c0, c1, c2, b2):
    b = m.shape[0]
    return pl.pallas_call(
        _head_body,
        out_shape=(
            jax.ShapeDtypeStruct((b, c2.shape[1]), jnp.float32),
            jax.ShapeDtypeStruct((b, 2 * c0.shape[1]), jnp.float32),
        ),
    )(m, v, s, c0, c1, c2, b2.reshape(1, -1))


def _global_idx(idx_t, n, b):
    g = jnp.swapaxes(idx_t, 1, 2)
    return g + (jnp.arange(b, dtype=jnp.int32) * n)[:, None, None]


def _chunked(idx_g, np_):
    return idx_g.reshape(-1, np_ * _K)


def _sub4(x, b, n):
    return x.reshape(b, n, -1)[:, ::4].reshape(b * n // 4, -1)


def kernel(xyz, params):
    b = xyz.shape[0]
    pts_full = jnp.swapaxes(xyz, 1, 2)
    h = b // 2
    halves = [pts_full[:h], pts_full[h:]]

    lw = params['lpfa_W']
    wsum = lw[0:3] + lw[3:6]
    wdiff = lw[6:9] - lw[0:3]

    def _w(p, cout, ratio):
        mid = cout // ratio
        return p['W1'], p['W2'][:mid], p['W2'][mid:] - p['W2'][:mid], p['Wsc']

    st = []
    for pts in halves:
        idxt1024 = _knn(pts, _K)
        idx1024 = _global_idx(idxt1024, 1024, h)
        idx256 = _global_idx(_knn(pts[:, ::4], _K), 256, h)
        idx64 = _global_idx(_knn(pts[:, ::16], _K), 64, h)
        chunks_for = {1024: _chunked(idx1024, 4),
                      256: _chunked(idx256, 2),
                      64: _chunked(idx64, 2)}
        sub_for = {1024: _chunked(idx1024[:, ::4], 4),
                   256: _chunked(idx256[:, ::4], 2)}
        feat = _lpfa(pts, idxt1024, wsum, wdiff)
        st.append(dict(chunks=chunks_for, subs=sub_for, cur_n=1024,
                       m=feat.reshape(h * 1024, 32), v=None, s=None))

    for bi in range(len(_CFGS)):
        (npoint, cin, cout, ratio), p = _CFGS[bi], params['cic'][bi]
        w1, w2a, w2d, wsc = _w(p, cout, ratio)
        nxt = _CFGS[bi + 1][0] if bi + 1 < len(_CFGS) else None
        for t in st:
            t['uvs'] = _tables(t['m'], t['v'], t['s'], w1, w2a, w2d, wsc)
        for t in st:
            u, vn, sn = t['uvs']
            if nxt is not None and nxt < t['cur_n']:
                t['m'] = _gmax(u, t['subs'][t['cur_n']], cout)
                t['v'] = _sub4(vn, h, t['cur_n'])
                t['s'] = _sub4(sn, h, t['cur_n'])
                t['cur_n'] = nxt
            else:
                t['m'] = _gmax(u, t['chunks'][t['cur_n']], cout)
                t['v'], t['s'] = vn, sn

    c0 = params['conv0_W']
    outs = [_head(t['m'].reshape(h, 64, 512), t['v'].reshape(h, 64, 512),
                  t['s'].reshape(h, 64, 512), c0, params['conv1_W'],
                  params['conv2_W'], params['conv2_b']) for t in st]
    return (jnp.concatenate([outs[0][0], outs[1][0]], axis=0),
            jnp.concatenate([outs[0][1], outs[1][1]], axis=0))
